# Initial kernel scaffold; baseline (speedup 1.0000x reference)
import jax
import jax.numpy as jnp
from jax.experimental import pallas as pl


def kernel(x, edge_index, edge_attr, W0, b0, W1, b1, W2, b2, W3, b3):
    raise NotImplementedError("write your pallas kernel here")


# speedup vs baseline: 3.9270x; 3.9270x over previous
import jax
import jax.numpy as jnp
from jax import lax
from jax.experimental import pallas as pl
from jax.experimental.pallas import tpu as pltpu
from jax.experimental.pallas import tpu_sc as plsc

N = 10000
NPAD = 10240
E = 160000
EPAD = 163840
NC = 2
NS = 16
NT = NC * NS
G = 40
B = 128
RPT = NPAD // NS
DUMMY = 10016

_mesh = plsc.VectorSubcoreMesh(core_axis_name="c", subcore_axis_name="s",
                               num_cores=NC, num_subcores=NS)


def _agg_body(table, src_hbm, dst_hbm, out,
              src_v, dst_v, buf0, buf1, zrow, acc, sem0, sem1):
    c = lax.axis_index("c")
    s = lax.axis_index("s")
    wid = c * NS + s
    base = s * RPT

    pltpu.sync_copy(src_hbm.at[wid], src_v)
    pltpu.sync_copy(dst_hbm.at[wid], dst_v)

    zero16 = jnp.zeros((16,), jnp.float32)

    def zbody(r, _):
        for j in range(8):
            zrow[r, pl.ds(j * 16, 16)] = zero16
        return 0
    lax.fori_loop(0, 16, zbody, 0)
    for i in range(RPT // 16):
        pltpu.sync_copy(zrow, acc.at[pl.ds(base + i * 16, 16)])
    plsc.subcore_barrier()

    def gbody(g, _):
        pltpu.sync_copy(table.at[src_v.at[g]], buf0)
        pltpu.sync_copy(buf0, acc.at[dst_v.at[g]], add=True)
        return 0
    lax.fori_loop(0, G, gbody, 0)

    plsc.subcore_barrier()
    for i in range(RPT // B):
        sl = pl.ds(base + i * B, B)
        pltpu.sync_copy(acc.at[sl], buf0)
        pltpu.sync_copy(buf0, out.at[c, sl])


_agg = pl.kernel(
    _agg_body,
    out_type=jax.ShapeDtypeStruct((NC, NPAD, 128), jnp.float32),
    mesh=_mesh,
    scratch_types=[
        pltpu.VMEM((G, B), jnp.int32),
        pltpu.VMEM((G, B), jnp.int32),
        pltpu.VMEM((B, 128), jnp.float32),
        pltpu.VMEM((B, 128), jnp.float32),
        pltpu.VMEM((16, 128), jnp.float32),
        pltpu.VMEM_SHARED((NPAD, 128), jnp.float32),
        pltpu.SemaphoreType.DMA,
        pltpu.SemaphoreType.DMA,
    ],
)


def _deg_body(dst_hbm, out, dst_v, ones_v, buf, zrow, acc):
    c = lax.axis_index("c")
    s = lax.axis_index("s")
    wid = c * NS + s
    base = s * RPT

    pltpu.sync_copy(dst_hbm.at[wid], dst_v)

    one16 = jnp.ones((16,), jnp.float32)
    zero16 = jnp.zeros((16,), jnp.float32)

    def obody(r, _):
        ones_v[r, pl.ds(0, 16)] = one16
        return 0
    lax.fori_loop(0, B, obody, 0)

    def zbody(r, _):
        zrow[r, pl.ds(0, 16)] = zero16
        return 0
    lax.fori_loop(0, 64, zbody, 0)
    for i in range(RPT // 64):
        pltpu.sync_copy(zrow, acc.at[pl.ds(base + i * 64, 64)])
    plsc.subcore_barrier()

    def gbody(g, _):
        pltpu.sync_copy(ones_v, acc.at[dst_v.at[g]], add=True)
        return 0
    lax.fori_loop(0, G, gbody, 0)

    plsc.subcore_barrier()
    for i in range(RPT // B):
        sl = pl.ds(base + i * B, B)
        pltpu.sync_copy(acc.at[sl], buf)
        pltpu.sync_copy(buf, out.at[c, sl])


_deg = pl.kernel(
    _deg_body,
    out_type=jax.ShapeDtypeStruct((NC, NPAD, 16), jnp.float32),
    mesh=_mesh,
    scratch_types=[
        pltpu.VMEM((G, B), jnp.int32),
        pltpu.VMEM((B, 16), jnp.float32),
        pltpu.VMEM((B, 16), jnp.float32),
        pltpu.VMEM((64, 16), jnp.float32),
        pltpu.VMEM_SHARED((NPAD, 16), jnp.float32),
    ],
)

_R = 512


def _dinv_of(degp):
    return lax.rsqrt(1.0 + degp[0, :, 0:1] + degp[1, :, 0:1])


def _scale_x_body(degp, x, out):
    out[...] = x[...] * _dinv_of(degp)


_scale_x = pl.pallas_call(
    _scale_x_body,
    grid=(NPAD // _R,),
    in_specs=[
        pl.BlockSpec((2, _R, 16), lambda i: (0, i, 0)),
        pl.BlockSpec((_R, 256), lambda i: (i, 0)),
    ],
    out_specs=pl.BlockSpec((_R, 256), lambda i: (i, 0)),
    out_shape=jax.ShapeDtypeStruct((NPAD, 256), jnp.float32),
)


def _layer0_body(degp, px, xs, w0, b0, w1, out):
    dinv = _dinv_of(degp)
    agg = jnp.concatenate([px[0, 0] + px[0, 1], px[1, 0] + px[1, 1]], axis=1)
    gcat = (agg + xs[...]) * dinv
    h = jnp.maximum(
        jnp.dot(gcat, w0[...], preferred_element_type=jnp.float32) + b0[...],
        0.0)
    z = jnp.dot(h, w1[...], preferred_element_type=jnp.float32) * dinv
    out[...] = z


_layer0 = pl.pallas_call(
    _layer0_body,
    grid=(NPAD // _R,),
    in_specs=[
        pl.BlockSpec((2, _R, 16), lambda i: (0, i, 0)),
        pl.BlockSpec((2, 2, _R, 128), lambda i: (0, 0, i, 0)),
        pl.BlockSpec((_R, 256), lambda i: (i, 0)),
        pl.BlockSpec((256, 512), lambda i: (0, 0)),
        pl.BlockSpec((1, 512), lambda i: (0, 0)),
        pl.BlockSpec((512, 512), lambda i: (0, 0)),
    ],
    out_specs=pl.BlockSpec((_R, 512), lambda i: (i, 0)),
    out_shape=jax.ShapeDtypeStruct((NPAD, 512), jnp.float32),
)


def _make_mid(dout):
    def body(degp, p, zs, b, w, out):
        dinv = _dinv_of(degp)
        agg = jnp.concatenate([p[cc, 0] + p[cc, 1] for cc in range(4)],
                              axis=1)
        h = jnp.maximum((agg + zs[...]) * dinv + b[...], 0.0)
        z = jnp.dot(h, w[...], preferred_element_type=jnp.float32) * dinv
        out[...] = z

    return pl.pallas_call(
        body,
        grid=(NPAD // _R,),
        in_specs=[
            pl.BlockSpec((2, _R, 16), lambda i: (0, i, 0)),
            pl.BlockSpec((4, 2, _R, 128), lambda i: (0, 0, i, 0)),
            pl.BlockSpec((_R, 512), lambda i: (i, 0)),
            pl.BlockSpec((1, 512), lambda i: (0, 0)),
            pl.BlockSpec((512, dout), lambda i: (0, 0)),
        ],
        out_specs=pl.BlockSpec((_R, dout), lambda i: (i, 0)),
        out_shape=jax.ShapeDtypeStruct((NPAD, dout), jnp.float32),
    )


_mid4 = _make_mid(512)
_mid1 = _make_mid(128)


def _final_body(degp, p, zs, b, out):
    o = (p[0] + p[1] + zs[...]) * _dinv_of(degp) + b[...]
    m = jnp.max(o, axis=1, keepdims=True)
    ssum = jnp.sum(jnp.exp(o - m), axis=1, keepdims=True)
    out[...] = o - m - jnp.log(ssum)


_final = pl.pallas_call(
    _final_body,
    grid=(NPAD // _R,),
    in_specs=[
        pl.BlockSpec((2, _R, 16), lambda i: (0, i, 0)),
        pl.BlockSpec((2, _R, 128), lambda i: (0, i, 0)),
        pl.BlockSpec((_R, 128), lambda i: (i, 0)),
        pl.BlockSpec((1, 128), lambda i: (0, 0)),
    ],
    out_specs=pl.BlockSpec((_R, 128), lambda i: (i, 0)),
    out_shape=jax.ShapeDtypeStruct((NPAD, 128), jnp.float32),
)


def _agg_chunks(z, n_chunks, src_t, dst_t):
    parts = []
    tok = jnp.float32(0.0)
    for cc in range(n_chunks):
        t = z[:, cc * 128:(cc + 1) * 128] + tok
        p = _agg(t, src_t, dst_t)
        tok = p[0, 0, 0] * 0.0
        parts.append(p)
    return jnp.stack(parts)


def kernel(x, edge_index, edge_attr, W0, b0, W1, b1, W2, b2, W3, b3):
    src = edge_index[0].astype(jnp.int32)
    dst = edge_index[1].astype(jnp.int32)
    pad = EPAD - E
    src_t = jnp.concatenate([src, jnp.zeros((pad,), jnp.int32)]
                            ).reshape(NT, G, B)
    dst_t = jnp.concatenate([dst, jnp.full((pad,), DUMMY, jnp.int32)]
                            ).reshape(NT, G, B)

    degp = _deg(dst_t)
    xp = jnp.pad(x, ((0, NPAD - N), (0, 0)))
    xs = _scale_x(degp, xp)

    px = _agg_chunks(xs + degp[0, 0, 0] * 0.0, 2, src_t, dst_t)
    z1s = _layer0(degp, px, xs, W0, b0.reshape(1, -1), W1)
    p1 = _agg_chunks(z1s + px[0, 0, 0, 0] * 0.0, 4, src_t, dst_t)
    z2s = _mid4(degp, p1, z1s, b1.reshape(1, -1), W2)
    p2 = _agg_chunks(z2s + p1[0, 0, 0, 0] * 0.0, 4, src_t, dst_t)
    z3s = _mid1(degp, p2, z2s, b2.reshape(1, -1), W3)
    p3 = _agg(z3s + p2[0, 0, 0, 0] * 0.0, src_t, dst_t)
    out = _final(degp, p3, z3s, b3.reshape(1, -1))
    return out[:N]


# speedup vs baseline: 4.0851x; 1.0403x over previous
import jax
import jax.numpy as jnp
from jax import lax
from jax.experimental import pallas as pl
from jax.experimental.pallas import tpu as pltpu
from jax.experimental.pallas import tpu_sc as plsc

N = 10000
NPAD = 10240
E = 160000
EPAD = 163840
NC = 2
NS = 16
NT = NC * NS
G = 40
B = 128
RPT = NPAD // NS
DUMMY = 10016

_mesh = plsc.VectorSubcoreMesh(core_axis_name="c", subcore_axis_name="s",
                               num_cores=NC, num_subcores=NS)


def _agg_body(table, src_hbm, dst_hbm, out,
              src_v, dst_v, buf0, buf1, zrow, acc, sem0, sem1):
    c = lax.axis_index("c")
    s = lax.axis_index("s")
    wid = c * NS + s
    base = s * RPT

    pltpu.sync_copy(src_hbm.at[wid], src_v)
    pltpu.sync_copy(dst_hbm.at[wid], dst_v)

    zero16 = jnp.zeros((16,), jnp.float32)

    def zbody(r, _):
        for j in range(8):
            zrow[r, pl.ds(j * 16, 16)] = zero16
        return 0
    lax.fori_loop(0, 16, zbody, 0)
    for i in range(RPT // 16):
        pltpu.sync_copy(zrow, acc.at[pl.ds(base + i * 16, 16)])
    plsc.subcore_barrier()

    def gbody(g, _):
        cp0 = pltpu.async_copy(table.at[src_v.at[2 * g]], buf0, sem0)
        cp1 = pltpu.async_copy(table.at[src_v.at[2 * g + 1]], buf1, sem1)
        cp0.wait()
        cp1.wait()
        pltpu.sync_copy(buf0, acc.at[dst_v.at[2 * g]], add=True)
        pltpu.sync_copy(buf1, acc.at[dst_v.at[2 * g + 1]], add=True)
        return 0
    lax.fori_loop(0, G // 2, gbody, 0)

    plsc.subcore_barrier()
    for i in range(RPT // B):
        sl = pl.ds(base + i * B, B)
        pltpu.sync_copy(acc.at[sl], buf0)
        pltpu.sync_copy(buf0, out.at[c, sl])


_agg = pl.kernel(
    _agg_body,
    out_type=jax.ShapeDtypeStruct((NC, NPAD, 128), jnp.float32),
    mesh=_mesh,
    scratch_types=[
        pltpu.VMEM((G, B), jnp.int32),
        pltpu.VMEM((G, B), jnp.int32),
        pltpu.VMEM((B, 128), jnp.float32),
        pltpu.VMEM((B, 128), jnp.float32),
        pltpu.VMEM((16, 128), jnp.float32),
        pltpu.VMEM_SHARED((NPAD, 128), jnp.float32),
        pltpu.SemaphoreType.DMA,
        pltpu.SemaphoreType.DMA,
    ],
)


def _deg_body(dst_hbm, out, dst_v, ones_v, buf, zrow, acc):
    c = lax.axis_index("c")
    s = lax.axis_index("s")
    wid = c * NS + s
    base = s * RPT

    pltpu.sync_copy(dst_hbm.at[wid], dst_v)

    one16 = jnp.ones((16,), jnp.float32)
    zero16 = jnp.zeros((16,), jnp.float32)

    def obody(r, _):
        ones_v[r, pl.ds(0, 16)] = one16
        return 0
    lax.fori_loop(0, B, obody, 0)

    def zbody(r, _):
        zrow[r, pl.ds(0, 16)] = zero16
        return 0
    lax.fori_loop(0, 64, zbody, 0)
    for i in range(RPT // 64):
        pltpu.sync_copy(zrow, acc.at[pl.ds(base + i * 64, 64)])
    plsc.subcore_barrier()

    def gbody(g, _):
        pltpu.sync_copy(ones_v, acc.at[dst_v.at[g]], add=True)
        return 0
    lax.fori_loop(0, G, gbody, 0)

    plsc.subcore_barrier()
    for i in range(RPT // B):
        sl = pl.ds(base + i * B, B)
        pltpu.sync_copy(acc.at[sl], buf)
        pltpu.sync_copy(buf, out.at[c, sl])


_deg = pl.kernel(
    _deg_body,
    out_type=jax.ShapeDtypeStruct((NC, NPAD, 16), jnp.float32),
    mesh=_mesh,
    scratch_types=[
        pltpu.VMEM((G, B), jnp.int32),
        pltpu.VMEM((B, 16), jnp.float32),
        pltpu.VMEM((B, 16), jnp.float32),
        pltpu.VMEM((64, 16), jnp.float32),
        pltpu.VMEM_SHARED((NPAD, 16), jnp.float32),
    ],
)

_R = 512


def _dinv_of(degp):
    return lax.rsqrt(1.0 + degp[0, :, 0:1] + degp[1, :, 0:1])


def _scale_x_body(degp, x, out):
    out[...] = x[...] * _dinv_of(degp)


_scale_x = pl.pallas_call(
    _scale_x_body,
    grid=(NPAD // _R,),
    in_specs=[
        pl.BlockSpec((2, _R, 16), lambda i: (0, i, 0)),
        pl.BlockSpec((_R, 256), lambda i: (i, 0)),
    ],
    out_specs=pl.BlockSpec((_R, 256), lambda i: (i, 0)),
    out_shape=jax.ShapeDtypeStruct((NPAD, 256), jnp.float32),
)


def _layer0_body(degp, px, xs, w0, b0, w1, out):
    dinv = _dinv_of(degp)
    agg = jnp.concatenate([px[0, 0] + px[0, 1], px[1, 0] + px[1, 1]], axis=1)
    gcat = (agg + xs[...]) * dinv
    h = jnp.maximum(
        jnp.dot(gcat, w0[...], preferred_element_type=jnp.float32) + b0[...],
        0.0)
    z = jnp.dot(h, w1[...], preferred_element_type=jnp.float32) * dinv
    out[...] = z


_layer0 = pl.pallas_call(
    _layer0_body,
    grid=(NPAD // _R,),
    in_specs=[
        pl.BlockSpec((2, _R, 16), lambda i: (0, i, 0)),
        pl.BlockSpec((2, 2, _R, 128), lambda i: (0, 0, i, 0)),
        pl.BlockSpec((_R, 256), lambda i: (i, 0)),
        pl.BlockSpec((256, 512), lambda i: (0, 0)),
        pl.BlockSpec((1, 512), lambda i: (0, 0)),
        pl.BlockSpec((512, 512), lambda i: (0, 0)),
    ],
    out_specs=pl.BlockSpec((_R, 512), lambda i: (i, 0)),
    out_shape=jax.ShapeDtypeStruct((NPAD, 512), jnp.float32),
)


def _make_mid(dout):
    def body(degp, p, zs, b, w, out):
        dinv = _dinv_of(degp)
        agg = jnp.concatenate([p[cc, 0] + p[cc, 1] for cc in range(4)],
                              axis=1)
        h = jnp.maximum((agg + zs[...]) * dinv + b[...], 0.0)
        z = jnp.dot(h, w[...], preferred_element_type=jnp.float32) * dinv
        out[...] = z

    return pl.pallas_call(
        body,
        grid=(NPAD // _R,),
        in_specs=[
            pl.BlockSpec((2, _R, 16), lambda i: (0, i, 0)),
            pl.BlockSpec((4, 2, _R, 128), lambda i: (0, 0, i, 0)),
            pl.BlockSpec((_R, 512), lambda i: (i, 0)),
            pl.BlockSpec((1, 512), lambda i: (0, 0)),
            pl.BlockSpec((512, dout), lambda i: (0, 0)),
        ],
        out_specs=pl.BlockSpec((_R, dout), lambda i: (i, 0)),
        out_shape=jax.ShapeDtypeStruct((NPAD, dout), jnp.float32),
    )


_mid4 = _make_mid(512)
_mid1 = _make_mid(128)


def _final_body(degp, p, zs, b, out):
    o = (p[0] + p[1] + zs[...]) * _dinv_of(degp) + b[...]
    m = jnp.max(o, axis=1, keepdims=True)
    ssum = jnp.sum(jnp.exp(o - m), axis=1, keepdims=True)
    out[...] = o - m - jnp.log(ssum)


_final = pl.pallas_call(
    _final_body,
    grid=(NPAD // _R,),
    in_specs=[
        pl.BlockSpec((2, _R, 16), lambda i: (0, i, 0)),
        pl.BlockSpec((2, _R, 128), lambda i: (0, i, 0)),
        pl.BlockSpec((_R, 128), lambda i: (i, 0)),
        pl.BlockSpec((1, 128), lambda i: (0, 0)),
    ],
    out_specs=pl.BlockSpec((_R, 128), lambda i: (i, 0)),
    out_shape=jax.ShapeDtypeStruct((NPAD, 128), jnp.float32),
)


def _agg_chunks(z, n_chunks, src_t, dst_t):
    parts = []
    tok = jnp.float32(0.0)
    for cc in range(n_chunks):
        t = z[:, cc * 128:(cc + 1) * 128] + tok
        p = _agg(t, src_t, dst_t)
        tok = p[0, 0, 0] * 0.0
        parts.append(p)
    return jnp.stack(parts)


def kernel(x, edge_index, edge_attr, W0, b0, W1, b1, W2, b2, W3, b3):
    src = edge_index[0].astype(jnp.int32)
    dst = edge_index[1].astype(jnp.int32)
    pad = EPAD - E
    src_t = jnp.concatenate([src, jnp.zeros((pad,), jnp.int32)]
                            ).reshape(NT, G, B)
    dst_t = jnp.concatenate([dst, jnp.full((pad,), DUMMY, jnp.int32)]
                            ).reshape(NT, G, B)

    degp = _deg(dst_t)
    xp = jnp.pad(x, ((0, NPAD - N), (0, 0)))
    xs = _scale_x(degp, xp)

    px = _agg_chunks(xs + degp[0, 0, 0] * 0.0, 2, src_t, dst_t)
    z1s = _layer0(degp, px, xs, W0, b0.reshape(1, -1), W1)
    p1 = _agg_chunks(z1s + px[0, 0, 0, 0] * 0.0, 4, src_t, dst_t)
    z2s = _mid4(degp, p1, z1s, b1.reshape(1, -1), W2)
    p2 = _agg_chunks(z2s + p1[0, 0, 0, 0] * 0.0, 4, src_t, dst_t)
    z3s = _mid1(degp, p2, z2s, b2.reshape(1, -1), W3)
    p3 = _agg(z3s + p2[0, 0, 0, 0] * 0.0, src_t, dst_t)
    out = _final(degp, p3, z3s, b3.reshape(1, -1))
    return out[:N]
